# Initial kernel scaffold; baseline (speedup 1.0000x reference)
import jax
import jax.numpy as jnp
from jax.experimental import pallas as pl


def kernel(cards, table):
    raise NotImplementedError("write your pallas kernel here")


# speedup vs baseline: 3.5036x; 3.5036x over previous
import functools

import jax
import jax.numpy as jnp
from jax import lax
from jax.experimental import pallas as pl
from jax.experimental.pallas import tpu as pltpu
from jax.experimental.pallas import tpu_sc as plsc

EMBEDDING_DIM = 32
VOCAB = 52


@functools.lru_cache(maxsize=None)
def _make_sc_gather(B: int, D: int, chunk: int):
    info = plsc.get_sparse_core_info()
    NC, NS = info.num_cores, info.num_subcores
    NW = NC * NS
    assert B % (NW * chunk) == 0
    b_per_w = B // NW
    n_chunks = b_per_w // chunk
    mesh = plsc.VectorSubcoreMesh(core_axis_name="c", subcore_axis_name="s")

    @functools.partial(
        pl.kernel,
        mesh=mesh,
        out_type=jax.ShapeDtypeStruct((B, D), jnp.float32),
        scratch_types=[
            pltpu.VMEM((chunk,), jnp.int32),
            pltpu.VMEM((chunk, D), jnp.float32),
            pltpu.SemaphoreType.DMA,
        ],
        compiler_params=pltpu.CompilerParams(use_tc_tiling_on_sc=False),
    )
    def sc_gather(idx_hbm, table_hbm, out_hbm, idx_v, rows_v, sem):
        wid = lax.axis_index("s") * NC + lax.axis_index("c")
        base0 = wid * b_per_w

        def body(i, carry):
            base = base0 + i * chunk
            pltpu.sync_copy(idx_hbm.at[pl.ds(base, chunk)], idx_v)
            pltpu.async_copy(table_hbm.at[idx_v], rows_v, sem).wait()
            pltpu.sync_copy(rows_v, out_hbm.at[pl.ds(base, chunk)])
            return carry

        lax.fori_loop(0, n_chunks, body, 0)

    return sc_gather


def kernel(cards, table):
    b, s = cards.shape
    B = b * s
    idx = cards.reshape(B).astype(jnp.int32)
    out = _make_sc_gather(B, EMBEDDING_DIM, 2048)(idx, table)
    return out.reshape(b, s, EMBEDDING_DIM)


# speedup vs baseline: 3.5051x; 1.0004x over previous
import functools

import jax
import jax.numpy as jnp
from jax import lax
from jax.experimental import pallas as pl
from jax.experimental.pallas import tpu as pltpu
from jax.experimental.pallas import tpu_sc as plsc

EMBEDDING_DIM = 32
VOCAB = 52
CHUNK = 1280
NBUF = 2


@functools.lru_cache(maxsize=None)
def _make_sc_gather(B: int, D: int):
    info = plsc.get_sparse_core_info()
    NC, NS = info.num_cores, info.num_subcores
    NW = NC * NS
    assert B % (NW * CHUNK) == 0
    b_per_w = B // NW
    n_chunks = b_per_w // CHUNK
    mesh = plsc.VectorSubcoreMesh(core_axis_name="c", subcore_axis_name="s")

    @functools.partial(
        pl.kernel,
        mesh=mesh,
        out_type=jax.ShapeDtypeStruct((B, D), jnp.float32),
        scratch_types=[
            pltpu.VMEM((n_chunks, CHUNK), jnp.int32),
            pltpu.VMEM((NBUF, CHUNK, D), jnp.float32),
            pltpu.SemaphoreType.DMA,
            pltpu.SemaphoreType.DMA,
            pltpu.SemaphoreType.DMA,
            pltpu.SemaphoreType.DMA,
        ],
        compiler_params=pltpu.CompilerParams(use_tc_tiling_on_sc=False),
    )
    def sc_gather(idx_hbm, table_hbm, out_hbm, idx_v, rows_v, gs0, gs1, ss0, ss1):
        wid = lax.axis_index("s") * NC + lax.axis_index("c")
        base0 = wid * b_per_w
        gsem = (gs0, gs1)
        ssem = (ss0, ss1)

        pltpu.sync_copy(idx_hbm.at[wid], idx_v)

        gathers = [None] * n_chunks
        stores = [None] * n_chunks
        for i in range(n_chunks):
            b = i % NBUF
            if i >= NBUF:
                stores[i - NBUF].wait()
            gathers[i] = pltpu.async_copy(
                table_hbm.at[idx_v.at[i]], rows_v.at[b], gsem[b]
            )
            if i >= 1:
                bb = (i - 1) % NBUF
                gathers[i - 1].wait()
                stores[i - 1] = pltpu.async_copy(
                    rows_v.at[bb],
                    out_hbm.at[pl.ds(base0 + (i - 1) * CHUNK, CHUNK)],
                    ssem[bb],
                )
        last = n_chunks - 1
        gathers[last].wait()
        stores[last] = pltpu.async_copy(
            rows_v.at[last % NBUF],
            out_hbm.at[pl.ds(base0 + last * CHUNK, CHUNK)],
            ssem[last % NBUF],
        )
        stores[last - 1].wait()
        stores[last].wait()

    return sc_gather


def kernel(cards, table):
    b, s = cards.shape
    B = b * s
    info = plsc.get_sparse_core_info()
    NW = info.num_cores * info.num_subcores
    idx = cards.reshape(NW, (B // NW) // CHUNK, CHUNK).astype(jnp.int32)
    out = _make_sc_gather(B, EMBEDDING_DIM)(idx, table)
    return out.reshape(b, s, EMBEDDING_DIM)
